# baseline (speedup 1.0000x reference)
import functools

import jax
import jax.numpy as jnp
from jax.experimental import pallas as pl
from jax.experimental.pallas import tpu as pltpu

B, D, P, A, M = 16384, 512, 256, 32, 4096
BETA_LONG, BETA_SHORT, ALPHA_LP = 0.995, 0.9, 0.5

ERR_TILE = 2048


def _err_body(obs_ref, nobs_ref, act_ref, we_ref, be_ref, wf1_ref, wf2_ref,
              bf_ref, err_ref):
    obs = obs_ref[...]
    nobs = nobs_ref[...]
    phi_t = jnp.maximum(obs @ we_ref[...] + be_ref[...], 0.0)
    phi_tp1 = jnp.maximum(nobs @ we_ref[...] + be_ref[...], 0.0)
    pred = phi_t @ wf1_ref[...] + act_ref[...] @ wf2_ref[...] + bf_ref[...]
    d = pred - phi_tp1
    err_ref[...] = jnp.sum(d * d, axis=1, keepdims=True) * (1.0 / P)


def _err_tc(obs, next_obs, actions, W_enc, b_enc, W_fwd, b_fwd):
    W_fwd1 = W_fwd[:P]
    W_fwd2 = W_fwd[P:]
    grid = B // ERR_TILE
    return pl.pallas_call(
        _err_body,
        grid=(grid,),
        in_specs=[
            pl.BlockSpec((ERR_TILE, D), lambda i: (i, 0)),
            pl.BlockSpec((ERR_TILE, D), lambda i: (i, 0)),
            pl.BlockSpec((ERR_TILE, A), lambda i: (i, 0)),
            pl.BlockSpec((D, P), lambda i: (0, 0)),
            pl.BlockSpec((1, P), lambda i: (0, 0)),
            pl.BlockSpec((P, P), lambda i: (0, 0)),
            pl.BlockSpec((A, P), lambda i: (0, 0)),
            pl.BlockSpec((1, P), lambda i: (0, 0)),
        ],
        out_specs=pl.BlockSpec((ERR_TILE, 1), lambda i: (i, 0)),
        out_shape=jax.ShapeDtypeStruct((B, 1), jnp.float32),
    )(obs, next_obs, actions, W_enc, b_enc.reshape(1, P), W_fwd1, W_fwd2,
      b_fwd.reshape(1, P))


def _scale_body(lp_ref, pms_ref, out_ref):
    lp = lp_ref[...]
    sumsq = jnp.sum(lp * lp)
    ms = 0.99 * pms_ref[0, 0] + 0.01 * sumsq * (1.0 / B)
    rms = jnp.sqrt(ms + 1e-8)
    out_ref[...] = (ALPHA_LP / (rms + 1e-8)) * lp


def _scale_tc(lp2d, prev_ms):
    return pl.pallas_call(
        _scale_body,
        in_specs=[
            pl.BlockSpec((B // 128, 128), lambda: (0, 0)),
            pl.BlockSpec((1, 1), lambda: (0, 0)),
        ],
        out_specs=pl.BlockSpec((B // 128, 128), lambda: (0, 0)),
        out_shape=jax.ShapeDtypeStruct((B // 128, 128), jnp.float32),
    )(lp2d, prev_ms.reshape(1, 1))


def _segment_xla(err, rids, ema_long, ema_short, counts):
    sums = jax.ops.segment_sum(err, rids, num_segments=M)
    cnts = jax.ops.segment_sum(jnp.ones_like(err), rids, num_segments=M)
    means = sums / jnp.maximum(1.0, cnts)
    present = cnts > 0
    is_new = counts == 0
    new_long = jnp.where(present, jnp.where(is_new, means,
                 BETA_LONG * ema_long + (1.0 - BETA_LONG) * means), ema_long)
    new_short = jnp.where(present, jnp.where(is_new, means,
                  BETA_SHORT * ema_short + (1.0 - BETA_SHORT) * means), ema_short)
    lp_region = jnp.where(present & (~is_new),
                          jnp.maximum(0.0, new_long - new_short),
                          jnp.zeros_like(new_long))
    return lp_region[rids]


def kernel(obs, next_obs, actions, rids, ema_long, ema_short, counts,
           W_enc, b_enc, W_fwd, b_fwd, prev_ms):
    err = _err_tc(obs, next_obs, actions, W_enc, b_enc, W_fwd, b_fwd)
    lp = _segment_xla(err[:, 0], rids, ema_long, ema_short, counts)
    out = _scale_tc(lp.reshape(B // 128, 128), prev_ms)
    return out.reshape(B)


# speedup vs baseline: 5.0816x; 5.0816x over previous
import functools

import jax
import jax.numpy as jnp
from jax import lax
from jax.experimental import pallas as pl
from jax.experimental.pallas import tpu as pltpu
from jax.experimental.pallas import tpu_sc as plsc

B, D, P, A, M = 16384, 512, 256, 32, 4096
BETA_LONG, BETA_SHORT, ALPHA_LP = 0.995, 0.9, 0.5

ERR_TILE = 2048


def _err_body(obs_ref, nobs_ref, act_ref, we_ref, be_ref, wf1_ref, wf2_ref,
              bf_ref, err_ref):
    obs = obs_ref[...]
    nobs = nobs_ref[...]
    phi_t = jnp.maximum(obs @ we_ref[...] + be_ref[...], 0.0)
    phi_tp1 = jnp.maximum(nobs @ we_ref[...] + be_ref[...], 0.0)
    pred = phi_t @ wf1_ref[...] + act_ref[...] @ wf2_ref[...] + bf_ref[...]
    d = pred - phi_tp1
    err_ref[...] = jnp.sum(d * d, axis=1, keepdims=True) * (1.0 / P)


def _err_tc(obs, next_obs, actions, W_enc, b_enc, W_fwd, b_fwd):
    W_fwd1 = W_fwd[:P]
    W_fwd2 = W_fwd[P:]
    grid = B // ERR_TILE
    return pl.pallas_call(
        _err_body,
        grid=(grid,),
        in_specs=[
            pl.BlockSpec((ERR_TILE, D), lambda i: (i, 0)),
            pl.BlockSpec((ERR_TILE, D), lambda i: (i, 0)),
            pl.BlockSpec((ERR_TILE, A), lambda i: (i, 0)),
            pl.BlockSpec((D, P), lambda i: (0, 0)),
            pl.BlockSpec((1, P), lambda i: (0, 0)),
            pl.BlockSpec((P, P), lambda i: (0, 0)),
            pl.BlockSpec((A, P), lambda i: (0, 0)),
            pl.BlockSpec((1, P), lambda i: (0, 0)),
        ],
        out_specs=pl.BlockSpec((ERR_TILE, 1), lambda i: (i, 0)),
        out_shape=jax.ShapeDtypeStruct((B, 1), jnp.float32),
    )(obs, next_obs, actions, W_enc, b_enc.reshape(1, P), W_fwd1, W_fwd2,
      b_fwd.reshape(1, P))


def _scale_body(lp_ref, pms_ref, out_ref):
    lp = lp_ref[...]
    sumsq = jnp.sum(lp * lp)
    ms = 0.99 * pms_ref[0, 0] + 0.01 * sumsq * (1.0 / B)
    rms = jnp.sqrt(ms + 1e-8)
    out_ref[...] = (ALPHA_LP / (rms + 1e-8)) * lp


def _scale_tc(lp2d, prev_ms):
    return pl.pallas_call(
        _scale_body,
        in_specs=[
            pl.BlockSpec((B // 128, 128), lambda: (0, 0)),
            pl.BlockSpec((1, 1), lambda: (0, 0)),
        ],
        out_specs=pl.BlockSpec((B // 128, 128), lambda: (0, 0)),
        out_shape=jax.ShapeDtypeStruct((B // 128, 128), jnp.float32),
    )(lp2d, prev_ms.reshape(1, 1))


NS = 16
ROWS = B // 128
RPW = ROWS // NS
MS = M // NS


def _sc_middle(err2d, rids2d, ema_long, ema_short, counts):
    mesh = plsc.VectorSubcoreMesh(core_axis_name="c", subcore_axis_name="s")

    @functools.partial(
        pl.kernel, mesh=mesh,
        out_type=jax.ShapeDtypeStruct((ROWS, 128), jnp.float32),
        scratch_types=[
            pltpu.VMEM((RPW, 128), jnp.int32),
            pltpu.VMEM((RPW, 128), jnp.float32),
            pltpu.VMEM((RPW, 128), jnp.float32),
            pltpu.VMEM((128,), jnp.float32),
            pltpu.VMEM((MS,), jnp.float32),
            pltpu.VMEM((MS,), jnp.float32),
            pltpu.VMEM((MS,), jnp.float32),
            pltpu.VMEM((MS,), jnp.float32),
            pltpu.VMEM((MS,), jnp.int32),
            pltpu.VMEM((MS,), jnp.float32),
            pltpu.VMEM_SHARED((M,), jnp.float32),
            pltpu.VMEM_SHARED((M,), jnp.float32),
            pltpu.VMEM_SHARED((M,), jnp.float32),
        ],
    )
    def k(err_hbm, rid_hbm, el_hbm, es_hbm, c0_hbm, lp_hbm,
          rid_v, err_v, lp_v, ones_v, sums_t, cnts_t, el_t, es_t, c0_t,
          lpr_t, sums_sh, cnts_sh, lpr_sh):
        cid = lax.axis_index("c")
        sid = lax.axis_index("s")

        @pl.when(cid == 0)
        def _work():
            for j in range(128 // NS):
                ones_v[pl.ds(j * NS, NS)] = jnp.full((NS,), 1.0, jnp.float32)
            for j in range(MS // NS):
                sums_t[pl.ds(j * NS, NS)] = jnp.zeros((NS,), jnp.float32)
            base_m = sid * MS
            pltpu.sync_copy(sums_t, sums_sh.at[pl.ds(base_m, MS)])
            pltpu.sync_copy(sums_t, cnts_sh.at[pl.ds(base_m, MS)])

            row0 = sid * RPW
            pltpu.sync_copy(rid_hbm.at[pl.ds(row0, RPW)], rid_v)
            pltpu.sync_copy(err_hbm.at[pl.ds(row0, RPW)], err_v)
            plsc.subcore_barrier()

            for r in range(RPW):
                pltpu.sync_copy(err_v.at[r], sums_sh.at[rid_v.at[r]],
                                add=True)
                pltpu.sync_copy(ones_v, cnts_sh.at[rid_v.at[r]], add=True)
            plsc.subcore_barrier()

            pltpu.sync_copy(sums_sh.at[pl.ds(base_m, MS)], sums_t)
            pltpu.sync_copy(cnts_sh.at[pl.ds(base_m, MS)], cnts_t)
            pltpu.sync_copy(el_hbm.at[pl.ds(base_m, MS)], el_t)
            pltpu.sync_copy(es_hbm.at[pl.ds(base_m, MS)], es_t)
            pltpu.sync_copy(c0_hbm.at[pl.ds(base_m, MS)], c0_t)
            for j in range(MS // NS):
                sl = pl.ds(j * NS, NS)
                s = sums_t[sl]
                c = cnts_t[sl]
                el = el_t[sl]
                es = es_t[sl]
                c0 = c0_t[sl].astype(jnp.float32)
                means = s / jnp.maximum(c, 1.0)
                pres = jnp.minimum(c, 1.0)
                old = jnp.minimum(c0, 1.0)
                ema_l = BETA_LONG * el + (1.0 - BETA_LONG) * means
                ema_s = BETA_SHORT * es + (1.0 - BETA_SHORT) * means
                upd_l = (1.0 - old) * means + old * ema_l
                upd_s = (1.0 - old) * means + old * ema_s
                nl = (1.0 - pres) * el + pres * upd_l
                nsh = (1.0 - pres) * es + pres * upd_s
                lpr_t[sl] = (pres * old) * jnp.maximum(nl - nsh, 0.0)
            pltpu.sync_copy(lpr_t, lpr_sh.at[pl.ds(base_m, MS)])
            plsc.subcore_barrier()

            for r in range(RPW):
                pltpu.sync_copy(lpr_sh.at[rid_v.at[r]], lp_v.at[r])
            pltpu.sync_copy(lp_v, lp_hbm.at[pl.ds(row0, RPW)])

    return k(err2d, rids2d, ema_long, ema_short, counts)


def kernel(obs, next_obs, actions, rids, ema_long, ema_short, counts,
           W_enc, b_enc, W_fwd, b_fwd, prev_ms):
    err = _err_tc(obs, next_obs, actions, W_enc, b_enc, W_fwd, b_fwd)
    lp2d = _sc_middle(err.reshape(ROWS, 128), rids.reshape(ROWS, 128),
                      ema_long, ema_short, counts)
    out = _scale_tc(lp2d, prev_ms)
    return out.reshape(B)
